# baseline (device time: 269830 ns/iter reference)
import jax
import jax.numpy as jnp
from jax import lax
from jax.experimental import pallas as pl
from jax.experimental.pallas import tpu as pltpu

N_DEV = 4
M_PER = 1024
K = 4096
N_SH = 2048
NT = 512
N_TILES = N_SH // NT
COMM_DT = jnp.bfloat16
EM, EN = 2048, 512


def _snap_e4m3(z):
    a = jnp.abs(z)
    bits = lax.bitcast_convert_type(a, jnp.int32)
    e = (bits >> 23) - 127
    e = jnp.maximum(e, -6)
    q = lax.bitcast_convert_type((e - 3 + 127) << 23, jnp.float32)
    c = q * 12582912.0
    snapped = jnp.minimum((a + c) - c, 448.0)
    return jnp.sign(z) * snapped


def _body(
    x_ref,
    w_ref,
    out_ref,
    h1r,
    h2r,
    p1l,
    p1r,
    p2,
    wbuf,
    ybuf,
    ebuf,
    avbuf,
    aall,
    amax_s,
    p1_ssem,
    p1_rsem,
    p2_ssem,
    p2_rsem,
    am_ssem,
    am_rsem,
    wld,
    yst,
    stg,
    lcp,
    el,
    es,
    credit,
):
    my = lax.axis_index("i")
    left = (my - 1) % N_DEV
    right = (my + 1) % N_DEV
    diag = (my + 2) % N_DEV

    barrier = pltpu.get_barrier_semaphore()
    for nbr in (left, right):
        pl.semaphore_signal(
            barrier, inc=1, device_id=(nbr,), device_id_type=pl.DeviceIdType.MESH
        )
    pl.semaphore_wait(barrier, 2)

    amax_s[0, 0] = 0.0

    stage = pltpu.make_async_copy(x_ref, p2, stg)
    stage.start()

    p1_to_r = pltpu.make_async_remote_copy(
        src_ref=x_ref,
        dst_ref=p1l,
        send_sem=p1_ssem.at[0],
        recv_sem=p1_rsem.at[0],
        device_id=(right,),
        device_id_type=pl.DeviceIdType.MESH,
    )
    p1_to_r.start()
    p1_to_l = pltpu.make_async_remote_copy(
        src_ref=x_ref,
        dst_ref=h1r,
        send_sem=p1_ssem.at[1],
        recv_sem=p1_rsem.at[1],
        device_id=(left,),
        device_id_type=pl.DeviceIdType.MESH,
    )
    p1_to_l.start()

    st = {"g": 0, "wl": [None, None], "ys": [None, None]}
    total_steps = N_DEV * N_TILES

    def wstart(g):
        s = g % 2
        t = g % N_TILES
        cp = pltpu.make_async_copy(
            w_ref.at[:, pl.ds(t * NT, NT)], wbuf.at[s], wld.at[s]
        )
        cp.start()
        st["wl"][s] = cp

    def gemm_chunk(xsrc, origin):
        r0 = origin * M_PER
        for t in range(N_TILES):
            g = st["g"]
            s = g % 2
            st["wl"][s].wait()
            if g + 1 < total_steps:
                wstart(g + 1)
            y = lax.dot_general(
                xsrc[...],
                wbuf[s],
                dimension_numbers=(((1,), (0,)), ((), ())),
                preferred_element_type=jnp.float32,
            )
            if st["ys"][s] is not None:
                st["ys"][s].wait()
            ybuf[s] = y
            amax_s[0, 0] = jnp.maximum(amax_s[0, 0], jnp.max(jnp.abs(y)))
            cp = pltpu.make_async_copy(
                ybuf.at[s],
                out_ref.at[pl.ds(r0, M_PER), pl.ds(t * NT, NT)],
                yst.at[s],
            )
            cp.start()
            st["ys"][s] = cp
            st["g"] += 1

    wstart(0)
    stage.wait()
    gemm_chunk(p2, my)
    for nbr in (left, right):
        pl.semaphore_signal(
            credit, inc=1, device_id=(nbr,), device_id_type=pl.DeviceIdType.MESH
        )

    p1_to_r.wait_recv()
    pl.semaphore_wait(credit, 2)
    p2_to_r = pltpu.make_async_remote_copy(
        src_ref=p1l.at[pl.ds(0, M_PER // 2)],
        dst_ref=p2.at[pl.ds(0, M_PER // 2)],
        send_sem=p2_ssem.at[0],
        recv_sem=p2_rsem.at[0],
        device_id=(right,),
        device_id_type=pl.DeviceIdType.MESH,
    )
    p2_to_r.start()
    gemm_chunk(p1l, left)

    p1_to_l.wait_recv()
    cp_h1 = pltpu.make_async_copy(h1r, p1r, lcp.at[0])
    cp_h1.start()
    p2_to_l = pltpu.make_async_remote_copy(
        src_ref=h1r.at[pl.ds(M_PER // 2, M_PER // 2)],
        dst_ref=h2r,
        send_sem=p2_ssem.at[1],
        recv_sem=p2_rsem.at[1],
        device_id=(left,),
        device_id_type=pl.DeviceIdType.MESH,
    )
    p2_to_l.start()
    cp_h1.wait()
    gemm_chunk(p1r, right)

    p2_to_r.wait_recv()
    p2_to_l.wait_recv()
    cp_h2 = pltpu.make_async_copy(
        h2r, p2.at[pl.ds(M_PER // 2, M_PER // 2)], lcp.at[1]
    )
    cp_h2.start()
    cp_h2.wait()
    gemm_chunk(p2, diag)

    for cp in st["ys"]:
        if cp is not None:
            cp.wait()
    p1_to_r.wait_send()
    p1_to_l.wait_send()
    p2_to_r.wait_send()
    p2_to_l.wait_send()

    avbuf[...] = jnp.full((8, 128), amax_s[0, 0], jnp.float32)
    am_rdmas = []
    for k in range(1, N_DEV):
        peer = (my + k) % N_DEV
        rd = pltpu.make_async_remote_copy(
            src_ref=avbuf,
            dst_ref=aall.at[k - 1],
            send_sem=am_ssem.at[k - 1],
            recv_sem=am_rsem.at[k - 1],
            device_id=(peer,),
            device_id_type=pl.DeviceIdType.MESH,
        )
        rd.start()
        am_rdmas.append(rd)
    for rd in am_rdmas:
        rd.wait_recv()
    for rd in am_rdmas:
        rd.wait_send()

    g_amax = jnp.maximum(amax_s[0, 0], jnp.max(aall[...]))
    scale = g_amax / 448.0

    tiles = [(i, j) for i in range(4096 // EM) for j in range(N_SH // EN)]
    ep = {"ld": [None, None], "stq": [None, None]}

    def eload(idx):
        i, j = tiles[idx]
        s = idx % 2
        if ep["stq"][s] is not None:
            ep["stq"][s].wait()
            ep["stq"][s] = None
        cp = pltpu.make_async_copy(
            out_ref.at[pl.ds(i * EM, EM), pl.ds(j * EN, EN)], ebuf.at[s], el.at[s]
        )
        cp.start()
        ep["ld"][s] = cp

    eload(0)
    for idx in range(len(tiles)):
        s = idx % 2
        ep["ld"][s].wait()
        if idx + 1 < len(tiles):
            eload(idx + 1)
        z = jnp.clip(ebuf[s] / scale, -448.0, 448.0)
        ebuf[s] = z.astype(jnp.float8_e4m3fn).astype(jnp.float32) * scale
        i, j = tiles[idx]
        cp = pltpu.make_async_copy(
            ebuf.at[s], out_ref.at[pl.ds(i * EM, EM), pl.ds(j * EN, EN)], es.at[s]
        )
        cp.start()
        ep["stq"][s] = cp
    for cp in ep["stq"]:
        if cp is not None:
            cp.wait()


def _fused(xb, wb):
    return pl.pallas_call(
        _body,
        out_shape=[
            jax.ShapeDtypeStruct((N_DEV * M_PER, N_SH), jnp.float32),
            jax.ShapeDtypeStruct((M_PER, K), COMM_DT),
            jax.ShapeDtypeStruct((M_PER // 2, K), COMM_DT),
        ],
        in_specs=[
            pl.BlockSpec(memory_space=pl.ANY),
            pl.BlockSpec(memory_space=pl.ANY),
        ],
        out_specs=[
            pl.BlockSpec(memory_space=pl.ANY),
            pl.BlockSpec(memory_space=pl.ANY),
            pl.BlockSpec(memory_space=pl.ANY),
        ],
        scratch_shapes=[
            pltpu.VMEM((M_PER, K), COMM_DT),
            pltpu.VMEM((M_PER, K), COMM_DT),
            pltpu.VMEM((M_PER, K), COMM_DT),
            pltpu.VMEM((2, K, NT), COMM_DT),
            pltpu.VMEM((2, M_PER, NT), jnp.float32),
            pltpu.VMEM((2, EM, EN), jnp.float32),
            pltpu.VMEM((8, 128), jnp.float32),
            pltpu.VMEM((3, 8, 128), jnp.float32),
            pltpu.SMEM((1, 1), jnp.float32),
            pltpu.SemaphoreType.DMA((2,)),
            pltpu.SemaphoreType.DMA((2,)),
            pltpu.SemaphoreType.DMA((2,)),
            pltpu.SemaphoreType.DMA((2,)),
            pltpu.SemaphoreType.DMA((3,)),
            pltpu.SemaphoreType.DMA((3,)),
            pltpu.SemaphoreType.DMA((2,)),
            pltpu.SemaphoreType.DMA((2,)),
            pltpu.SemaphoreType.DMA,
            pltpu.SemaphoreType.DMA((2,)),
            pltpu.SemaphoreType.DMA((2,)),
            pltpu.SemaphoreType.DMA((2,)),
            pltpu.SemaphoreType.REGULAR,
        ],
        compiler_params=pltpu.CompilerParams(
            collective_id=0, vmem_limit_bytes=64 * 1024 * 1024
        ),
    )(xb, wb)


def kernel(x, w_mat):
    out, _h1, _h2 = _fused(x.astype(COMM_DT), w_mat.astype(COMM_DT))
    return out


# device time: 261378 ns/iter; 1.0323x vs baseline; 1.0323x over previous
import jax
import jax.numpy as jnp
from jax import lax
from jax.experimental import pallas as pl
from jax.experimental.pallas import tpu as pltpu

N_DEV = 4
M_PER = 1024
K = 4096
N_SH = 2048
NT = 512
N_TILES = N_SH // NT
COMM_DT = jnp.bfloat16
EM, EN = 2048, 512


def _snap_e4m3(z):
    a = jnp.abs(z)
    bits = lax.bitcast_convert_type(a, jnp.int32)
    e = (bits >> 23) - 127
    e = jnp.maximum(e, -6)
    q = lax.bitcast_convert_type((e - 3 + 127) << 23, jnp.float32)
    c = q * 12582912.0
    snapped = jnp.minimum((a + c) - c, 448.0)
    return jnp.sign(z) * snapped


def _body(
    x_ref,
    w_ref,
    out_ref,
    p1l,
    p1r,
    p2,
    wbuf,
    ybuf,
    ebuf,
    avbuf,
    aall,
    amax_s,
    p1_ssem,
    p1_rsem,
    p2_ssem,
    p2_rsem,
    am_ssem,
    am_rsem,
    wld,
    yst,
    stg,
    el,
    es,
    credit,
):
    my = lax.axis_index("i")
    left = (my - 1) % N_DEV
    right = (my + 1) % N_DEV
    diag = (my + 2) % N_DEV

    barrier = pltpu.get_barrier_semaphore()
    for nbr in (left, right):
        pl.semaphore_signal(
            barrier, inc=1, device_id=(nbr,), device_id_type=pl.DeviceIdType.MESH
        )
    pl.semaphore_wait(barrier, 2)

    amax_s[0, 0] = 0.0

    stage = pltpu.make_async_copy(x_ref, p2, stg)
    stage.start()

    p1_to_r = pltpu.make_async_remote_copy(
        src_ref=x_ref,
        dst_ref=p1l,
        send_sem=p1_ssem.at[0],
        recv_sem=p1_rsem.at[0],
        device_id=(right,),
        device_id_type=pl.DeviceIdType.MESH,
    )
    p1_to_r.start()
    p1_to_l = pltpu.make_async_remote_copy(
        src_ref=x_ref,
        dst_ref=p1r,
        send_sem=p1_ssem.at[1],
        recv_sem=p1_rsem.at[1],
        device_id=(left,),
        device_id_type=pl.DeviceIdType.MESH,
    )
    p1_to_l.start()

    st = {"g": 0, "wl": [None, None], "ys": [None, None]}
    total_steps = N_DEV * N_TILES

    def wstart(g):
        s = g % 2
        t = g % N_TILES
        cp = pltpu.make_async_copy(
            w_ref.at[:, pl.ds(t * NT, NT)], wbuf.at[s], wld.at[s]
        )
        cp.start()
        st["wl"][s] = cp

    def gemm_chunk(xsrc, origin):
        r0 = origin * M_PER
        for t in range(N_TILES):
            g = st["g"]
            s = g % 2
            st["wl"][s].wait()
            if g + 1 < total_steps:
                wstart(g + 1)
            y = lax.dot_general(
                xsrc[...],
                wbuf[s],
                dimension_numbers=(((1,), (0,)), ((), ())),
                preferred_element_type=jnp.float32,
            )
            if st["ys"][s] is not None:
                st["ys"][s].wait()
            ybuf[s] = y
            amax_s[0, 0] = jnp.maximum(amax_s[0, 0], jnp.max(jnp.abs(y)))
            cp = pltpu.make_async_copy(
                ybuf.at[s],
                out_ref.at[pl.ds(r0, M_PER), pl.ds(t * NT, NT)],
                yst.at[s],
            )
            cp.start()
            st["ys"][s] = cp
            st["g"] += 1

    wstart(0)
    stage.wait()
    gemm_chunk(p2, my)
    for nbr in (left, right):
        pl.semaphore_signal(
            credit, inc=1, device_id=(nbr,), device_id_type=pl.DeviceIdType.MESH
        )

    p1_to_r.wait_recv()
    pl.semaphore_wait(credit, 2)
    p2_to_r = pltpu.make_async_remote_copy(
        src_ref=p1l.at[pl.ds(0, M_PER // 2)],
        dst_ref=p2.at[pl.ds(0, M_PER // 2)],
        send_sem=p2_ssem.at[0],
        recv_sem=p2_rsem.at[0],
        device_id=(right,),
        device_id_type=pl.DeviceIdType.MESH,
    )
    p2_to_r.start()
    gemm_chunk(p1l, left)

    p1_to_l.wait_recv()
    p2_to_l = pltpu.make_async_remote_copy(
        src_ref=p1r.at[pl.ds(M_PER // 2, M_PER // 2)],
        dst_ref=p2.at[pl.ds(M_PER // 2, M_PER // 2)],
        send_sem=p2_ssem.at[1],
        recv_sem=p2_rsem.at[1],
        device_id=(left,),
        device_id_type=pl.DeviceIdType.MESH,
    )
    p2_to_l.start()
    gemm_chunk(p1r, right)

    p2_to_r.wait_recv()
    p2_to_l.wait_recv()
    gemm_chunk(p2, diag)

    for cp in st["ys"]:
        if cp is not None:
            cp.wait()
    p1_to_r.wait_send()
    p1_to_l.wait_send()
    p2_to_r.wait_send()
    p2_to_l.wait_send()

    avbuf[...] = jnp.full((8, 128), amax_s[0, 0], jnp.float32)
    am_rdmas = []
    for k in range(1, N_DEV):
        peer = (my + k) % N_DEV
        rd = pltpu.make_async_remote_copy(
            src_ref=avbuf,
            dst_ref=aall.at[k - 1],
            send_sem=am_ssem.at[k - 1],
            recv_sem=am_rsem.at[k - 1],
            device_id=(peer,),
            device_id_type=pl.DeviceIdType.MESH,
        )
        rd.start()
        am_rdmas.append(rd)

    tiles = [(i, j) for i in range(4096 // EM) for j in range(N_SH // EN)]
    ep = {"ld": [None, None], "stq": [None, None]}

    def eload(idx):
        i, j = tiles[idx]
        s = idx % 2
        if ep["stq"][s] is not None:
            ep["stq"][s].wait()
            ep["stq"][s] = None
        cp = pltpu.make_async_copy(
            out_ref.at[pl.ds(i * EM, EM), pl.ds(j * EN, EN)], ebuf.at[s], el.at[s]
        )
        cp.start()
        ep["ld"][s] = cp

    eload(0)
    eload(1)

    for rd in am_rdmas:
        rd.wait_recv()
    for rd in am_rdmas:
        rd.wait_send()

    g_amax = jnp.maximum(amax_s[0, 0], jnp.max(aall[...]))
    scale = g_amax / 448.0

    for idx in range(len(tiles)):
        s = idx % 2
        ep["ld"][s].wait()
        z = jnp.clip(ebuf[s] / scale, -448.0, 448.0)
        ebuf[s] = z.astype(jnp.float8_e4m3fn).astype(jnp.float32) * scale
        i, j = tiles[idx]
        cp = pltpu.make_async_copy(
            ebuf.at[s], out_ref.at[pl.ds(i * EM, EM), pl.ds(j * EN, EN)], es.at[s]
        )
        cp.start()
        ep["stq"][s] = cp
        if idx + 2 < len(tiles):
            eload(idx + 2)
    for cp in ep["stq"]:
        if cp is not None:
            cp.wait()


def _fused(xb, wb):
    return pl.pallas_call(
        _body,
        out_shape=jax.ShapeDtypeStruct((N_DEV * M_PER, N_SH), jnp.float32),
        in_specs=[
            pl.BlockSpec(memory_space=pl.ANY),
            pl.BlockSpec(memory_space=pl.ANY),
        ],
        out_specs=pl.BlockSpec(memory_space=pl.ANY),
        scratch_shapes=[
            pltpu.VMEM((M_PER, K), COMM_DT),
            pltpu.VMEM((M_PER, K), COMM_DT),
            pltpu.VMEM((M_PER, K), COMM_DT),
            pltpu.VMEM((2, K, NT), COMM_DT),
            pltpu.VMEM((2, M_PER, NT), jnp.float32),
            pltpu.VMEM((2, EM, EN), jnp.float32),
            pltpu.VMEM((8, 128), jnp.float32),
            pltpu.VMEM((3, 8, 128), jnp.float32),
            pltpu.SMEM((1, 1), jnp.float32),
            pltpu.SemaphoreType.DMA((2,)),
            pltpu.SemaphoreType.DMA((2,)),
            pltpu.SemaphoreType.DMA((2,)),
            pltpu.SemaphoreType.DMA((2,)),
            pltpu.SemaphoreType.DMA((3,)),
            pltpu.SemaphoreType.DMA((3,)),
            pltpu.SemaphoreType.DMA((2,)),
            pltpu.SemaphoreType.DMA((2,)),
            pltpu.SemaphoreType.DMA,
            pltpu.SemaphoreType.DMA((2,)),
            pltpu.SemaphoreType.DMA((2,)),
            pltpu.SemaphoreType.REGULAR,
        ],
        compiler_params=pltpu.CompilerParams(
            collective_id=0, vmem_limit_bytes=64 * 1024 * 1024
        ),
    )(xb, wb)


def kernel(x, w_mat):
    return _fused(x.astype(COMM_DT), w_mat.astype(COMM_DT))


# device time: 250787 ns/iter; 1.0759x vs baseline; 1.0422x over previous
import jax
import jax.numpy as jnp
from jax import lax
from jax.experimental import pallas as pl
from jax.experimental.pallas import tpu as pltpu

N_DEV = 4
M_PER = 1024
K = 4096
N_SH = 2048
NT = 512
N_TILES = N_SH // NT
COMM_DT = jnp.bfloat16
EM, EN = 2048, 512


def _snap_e4m3(z):
    a = jnp.abs(z)
    bits = lax.bitcast_convert_type(a, jnp.int32)
    e = (bits >> 23) - 127
    e = jnp.maximum(e, -6)
    q = lax.bitcast_convert_type((e - 3 + 127) << 23, jnp.float32)
    c = q * 12582912.0
    snapped = jnp.minimum((a + c) - c, 448.0)
    return jnp.sign(z) * snapped


def _body(
    x_ref,
    w_ref,
    out_ref,
    p1l,
    p1r,
    p2,
    wbuf,
    ybuf,
    ebuf,
    avbuf,
    aall,
    amax_s,
    p1_ssem,
    p1_rsem,
    p2_ssem,
    p2_rsem,
    am_ssem,
    am_rsem,
    wld,
    yst,
    stg,
    el,
    es,
    credit,
):
    my = lax.axis_index("i")
    left = (my - 1) % N_DEV
    right = (my + 1) % N_DEV
    diag = (my + 2) % N_DEV

    barrier = pltpu.get_barrier_semaphore()
    for nbr in (left, right):
        pl.semaphore_signal(
            barrier, inc=1, device_id=(nbr,), device_id_type=pl.DeviceIdType.MESH
        )
    pl.semaphore_wait(barrier, 2)

    amax_s[0, 0] = 0.0

    stage = pltpu.make_async_copy(x_ref, p2, stg)
    stage.start()

    p1_to_r = pltpu.make_async_remote_copy(
        src_ref=x_ref,
        dst_ref=p1l,
        send_sem=p1_ssem.at[0],
        recv_sem=p1_rsem.at[0],
        device_id=(right,),
        device_id_type=pl.DeviceIdType.MESH,
    )
    p1_to_r.start()
    p1_to_l = pltpu.make_async_remote_copy(
        src_ref=x_ref,
        dst_ref=p1r,
        send_sem=p1_ssem.at[1],
        recv_sem=p1_rsem.at[1],
        device_id=(left,),
        device_id_type=pl.DeviceIdType.MESH,
    )
    p1_to_l.start()

    st = {"g": 0, "wl": [None, None], "ys": [None, None]}
    total_steps = (3 + 4) * N_TILES

    def wstart(g):
        s = g % 2
        t = g % N_TILES
        cp = pltpu.make_async_copy(
            w_ref.at[:, pl.ds(t * NT, NT)], wbuf.at[s], wld.at[s]
        )
        cp.start()
        st["wl"][s] = cp

    def gemm_chunk(xsrc, origin, row0=0, rows=M_PER):
        r0 = origin * M_PER + row0
        for t in range(N_TILES):
            g = st["g"]
            s = g % 2
            st["wl"][s].wait()
            if g + 1 < total_steps:
                wstart(g + 1)
            y = lax.dot_general(
                xsrc[row0 : row0 + rows, :],
                wbuf[s],
                dimension_numbers=(((1,), (0,)), ((), ())),
                preferred_element_type=jnp.float32,
            )
            if st["ys"][s] is not None:
                st["ys"][s].wait()
            ybuf[s, 0:rows] = y
            amax_s[0, 0] = jnp.maximum(amax_s[0, 0], jnp.max(jnp.abs(y)))
            cp = pltpu.make_async_copy(
                ybuf.at[s, pl.ds(0, rows)],
                out_ref.at[pl.ds(r0, rows), pl.ds(t * NT, NT)],
                yst.at[s],
            )
            cp.start()
            st["ys"][s] = cp
            st["g"] += 1

    wstart(0)
    stage.wait()
    gemm_chunk(p2, my)
    for nbr in (left, right):
        pl.semaphore_signal(
            credit, inc=1, device_id=(nbr,), device_id_type=pl.DeviceIdType.MESH
        )

    Q = M_PER // 4

    p1_to_r.wait_recv()
    pl.semaphore_wait(credit, 2)
    p2_rdmas = []
    for q in range(2):
        rd = pltpu.make_async_remote_copy(
            src_ref=p1l.at[pl.ds(q * Q, Q)],
            dst_ref=p2.at[pl.ds(q * Q, Q)],
            send_sem=p2_ssem.at[q],
            recv_sem=p2_rsem.at[q],
            device_id=(right,),
            device_id_type=pl.DeviceIdType.MESH,
        )
        rd.start()
        p2_rdmas.append(rd)
    gemm_chunk(p1l, left)

    p1_to_l.wait_recv()
    for q in range(2, 4):
        rd = pltpu.make_async_remote_copy(
            src_ref=p1r.at[pl.ds(q * Q, Q)],
            dst_ref=p2.at[pl.ds(q * Q, Q)],
            send_sem=p2_ssem.at[q],
            recv_sem=p2_rsem.at[q],
            device_id=(left,),
            device_id_type=pl.DeviceIdType.MESH,
        )
        rd.start()
        p2_rdmas.append(rd)
    gemm_chunk(p1r, right)

    for q in range(4):
        p2_rdmas[q].wait_recv()
        gemm_chunk(p2, diag, row0=q * Q, rows=Q)

    for cp in st["ys"]:
        if cp is not None:
            cp.wait()
    p1_to_r.wait_send()
    p1_to_l.wait_send()
    for rd in p2_rdmas:
        rd.wait_send()

    avbuf[...] = jnp.full((8, 128), amax_s[0, 0], jnp.float32)
    am_rdmas = []
    for k in range(1, N_DEV):
        peer = (my + k) % N_DEV
        rd = pltpu.make_async_remote_copy(
            src_ref=avbuf,
            dst_ref=aall.at[k - 1],
            send_sem=am_ssem.at[k - 1],
            recv_sem=am_rsem.at[k - 1],
            device_id=(peer,),
            device_id_type=pl.DeviceIdType.MESH,
        )
        rd.start()
        am_rdmas.append(rd)

    tiles = [(i, j) for i in range(4096 // EM) for j in range(N_SH // EN)]
    ep = {"ld": [None, None], "stq": [None, None]}

    def eload(idx):
        i, j = tiles[idx]
        s = idx % 2
        if ep["stq"][s] is not None:
            ep["stq"][s].wait()
            ep["stq"][s] = None
        cp = pltpu.make_async_copy(
            out_ref.at[pl.ds(i * EM, EM), pl.ds(j * EN, EN)], ebuf.at[s], el.at[s]
        )
        cp.start()
        ep["ld"][s] = cp

    eload(0)
    eload(1)

    for rd in am_rdmas:
        rd.wait_recv()
    for rd in am_rdmas:
        rd.wait_send()

    g_amax = jnp.maximum(amax_s[0, 0], jnp.max(aall[...]))
    scale = g_amax / 448.0

    for idx in range(len(tiles)):
        s = idx % 2
        ep["ld"][s].wait()
        z = jnp.clip(ebuf[s] / scale, -448.0, 448.0)
        ebuf[s] = z.astype(jnp.float8_e4m3fn).astype(jnp.float32) * scale
        i, j = tiles[idx]
        cp = pltpu.make_async_copy(
            ebuf.at[s], out_ref.at[pl.ds(i * EM, EM), pl.ds(j * EN, EN)], es.at[s]
        )
        cp.start()
        ep["stq"][s] = cp
        if idx + 2 < len(tiles):
            eload(idx + 2)
    for cp in ep["stq"]:
        if cp is not None:
            cp.wait()


def _fused(xb, wb):
    return pl.pallas_call(
        _body,
        out_shape=jax.ShapeDtypeStruct((N_DEV * M_PER, N_SH), jnp.float32),
        in_specs=[
            pl.BlockSpec(memory_space=pl.ANY),
            pl.BlockSpec(memory_space=pl.ANY),
        ],
        out_specs=pl.BlockSpec(memory_space=pl.ANY),
        scratch_shapes=[
            pltpu.VMEM((M_PER, K), COMM_DT),
            pltpu.VMEM((M_PER, K), COMM_DT),
            pltpu.VMEM((M_PER, K), COMM_DT),
            pltpu.VMEM((2, K, NT), COMM_DT),
            pltpu.VMEM((2, M_PER, NT), jnp.float32),
            pltpu.VMEM((2, EM, EN), jnp.float32),
            pltpu.VMEM((8, 128), jnp.float32),
            pltpu.VMEM((3, 8, 128), jnp.float32),
            pltpu.SMEM((1, 1), jnp.float32),
            pltpu.SemaphoreType.DMA((2,)),
            pltpu.SemaphoreType.DMA((2,)),
            pltpu.SemaphoreType.DMA((4,)),
            pltpu.SemaphoreType.DMA((4,)),
            pltpu.SemaphoreType.DMA((3,)),
            pltpu.SemaphoreType.DMA((3,)),
            pltpu.SemaphoreType.DMA((2,)),
            pltpu.SemaphoreType.DMA((2,)),
            pltpu.SemaphoreType.DMA,
            pltpu.SemaphoreType.DMA((2,)),
            pltpu.SemaphoreType.DMA((2,)),
            pltpu.SemaphoreType.REGULAR,
        ],
        compiler_params=pltpu.CompilerParams(
            collective_id=0, vmem_limit_bytes=64 * 1024 * 1024
        ),
    )(xb, wb)


def kernel(x, w_mat):
    return _fused(x.astype(COMM_DT), w_mat.astype(COMM_DT))
